# baseline (speedup 1.0000x reference)
import functools

import jax
import jax.numpy as jnp
from jax import lax
from jax.experimental import pallas as pl
from jax.experimental.pallas import tpu as pltpu
from jax.experimental.pallas import tpu_sc as plsc

_V = 100000
_D = 32
_TOKENS = 512

_NC = 2
_NS = 16
_NW = _NC * _NS
_TOK_PER_W = _TOKENS // _NW

_VT = 2048


def _make_sc_gather():
    mesh = plsc.VectorSubcoreMesh(core_axis_name="c", subcore_axis_name="s")

    @functools.partial(
        pl.kernel,
        mesh=mesh,
        compiler_params=pltpu.CompilerParams(use_tc_tiling_on_sc=False),
        out_type=jax.ShapeDtypeStruct((_TOKENS, _D), jnp.float32),
        scratch_types=[
            pltpu.VMEM((_TOK_PER_W,), jnp.int32),
            pltpu.VMEM((_TOK_PER_W, _D), jnp.float32),
            pltpu.SemaphoreType.DMA,
        ],
    )
    def gather_kernel(table_hbm, idx_hbm, out_hbm, idx_v, rows_v, sem):
        wid = lax.axis_index("s") * _NC + lax.axis_index("c")
        base = wid * _TOK_PER_W
        pltpu.sync_copy(idx_hbm.at[pl.ds(base, _TOK_PER_W)], idx_v)
        pltpu.async_copy(table_hbm.at[idx_v], rows_v, sem).wait()
        pltpu.sync_copy(rows_v, out_hbm.at[pl.ds(base, _TOK_PER_W)])

    return gather_kernel


_sc_gather = _make_sc_gather()


def _mm_body(x_ref, w_ref, b_ref, o_ref):
    o_ref[...] = (
        lax.dot_general(
            x_ref[...],
            w_ref[...],
            (((1,), (1,)), ((), ())),
            preferred_element_type=jnp.float32,
        )
        + b_ref[...]
    )


def kernel(input_ids, emb, W, b):
    B, S = input_ids.shape
    ids = input_ids.reshape(-1).astype(jnp.int32)
    x = _sc_gather(emb, ids)

    logits = pl.pallas_call(
        _mm_body,
        grid=(pl.cdiv(_V, _VT),),
        in_specs=[
            pl.BlockSpec((_TOKENS, _D), lambda i: (0, 0)),
            pl.BlockSpec((_VT, _D), lambda i: (i, 0)),
            pl.BlockSpec((1, _VT), lambda i: (0, i)),
        ],
        out_specs=pl.BlockSpec((_TOKENS, _VT), lambda i: (0, i)),
        out_shape=jax.ShapeDtypeStruct((_TOKENS, _V), jnp.float32),
    )(x, W, b.reshape(1, _V))

    return logits.reshape(B, S, _V)


# speedup vs baseline: 1.7397x; 1.7397x over previous
import jax
import jax.numpy as jnp
from jax import lax
from jax.experimental import pallas as pl
from jax.experimental.pallas import tpu as pltpu

_V = 100000
_D = 32
_B = 4
_S = 128
_VT = 4096


def _body(idx_ref, et_ref, wt_ref, b_ref, o_ref, xaug_s):
    iv = pl.program_id(0)
    pb = pl.program_id(1)

    @pl.when(jnp.logical_and(iv == 0, pb == 0))
    def _gather():
        xaug_s[:, _D:, :] = jnp.ones((_B, 1, _S), jnp.bfloat16)
        lane_iota = lax.broadcasted_iota(jnp.int32, (1, _S), 1)
        for bb in range(_B):

            def _tok(s, acc):
                idx = idx_ref[bb, s]
                win = et_ref[:, pl.ds((idx // _S) * _S, _S)]
                lane = jnp.full((_D, 1), idx % _S, jnp.int32)
                col = jnp.take_along_axis(win, lane, axis=1)
                colb = jnp.broadcast_to(col, (_D, _S))
                return jnp.where(lane_iota == s, colb, acc)

            acc = lax.fori_loop(0, _S, _tok, jnp.zeros((_D, _S), jnp.float32))
            xaug_s[bb, 0:_D, :] = acc.astype(jnp.bfloat16)

    lhs = jnp.concatenate([wt_ref[...], b_ref[...]], axis=0)
    o_ref[0] = lax.dot_general(
        lhs.astype(jnp.bfloat16),
        xaug_s[pb],
        (((0,), (0,)), ((), ())),
        preferred_element_type=jnp.float32,
    )


def kernel(input_ids, emb, W, b):
    ids = input_ids.astype(jnp.int32)
    embT = emb.T
    WT = W.T
    b1 = b.reshape(1, _V)

    t = pl.pallas_call(
        _body,
        grid=(pl.cdiv(_V, _VT), _B),
        in_specs=[
            pl.BlockSpec(memory_space=pltpu.SMEM),
            pl.BlockSpec((_D, _V), lambda i, pb: (0, 0)),
            pl.BlockSpec((_D, _VT), lambda i, pb: (0, i)),
            pl.BlockSpec((1, _VT), lambda i, pb: (0, i)),
        ],
        out_specs=pl.BlockSpec((1, _VT, _S), lambda i, pb: (pb, i, 0)),
        out_shape=jax.ShapeDtypeStruct((_B, _V, _S), jnp.float32),
        scratch_shapes=[pltpu.VMEM((_B, _D + 1, _S), jnp.bfloat16)],
    )(ids, embT, WT, b1)

    return t.transpose(0, 2, 1)


# speedup vs baseline: 2.0946x; 1.2040x over previous
import jax
import jax.numpy as jnp
from jax import lax
from jax.experimental import pallas as pl
from jax.experimental.pallas import tpu as pltpu

_V = 100000
_D = 32
_B = 4
_S = 128
_T = _B * _S
_VT = 2048


def _body(idx_ref, et_ref, wt_ref, b_ref, o_ref, xaug_s):
    @pl.when(pl.program_id(0) == 0)
    def _gather():
        xaug_s[_D:, :] = jnp.ones((1, _T), jnp.bfloat16)
        lane_iota = lax.broadcasted_iota(jnp.int32, (1, _T), 1)

        def _tok(t, acc):
            idx = idx_ref[t // _S, t % _S]
            win = et_ref[:, pl.ds((idx // _S) * _S, _S)]
            lane = jnp.full((_D, 1), idx % _S, jnp.int32)
            col = jnp.take_along_axis(win, lane, axis=1)
            colb = jnp.broadcast_to(col, (_D, _T))
            return jnp.where(lane_iota == t, colb, acc)

        acc = lax.fori_loop(0, _T, _tok, jnp.zeros((_D, _T), jnp.float32))
        xaug_s[0:_D, :] = acc.astype(jnp.bfloat16)

    lhs = jnp.concatenate([wt_ref[...], b_ref[...]], axis=0)
    res = lax.dot_general(
        lhs.astype(jnp.bfloat16),
        xaug_s[...],
        (((0,), (0,)), ((), ())),
        preferred_element_type=jnp.float32,
    )
    for bb in range(_B):
        o_ref[bb] = res[:, bb * _S:(bb + 1) * _S]


def kernel(input_ids, emb, W, b):
    ids = input_ids.astype(jnp.int32)
    embT = emb.T
    WT = W.T
    b1 = b.reshape(1, _V)

    t = pl.pallas_call(
        _body,
        grid=(pl.cdiv(_V, _VT),),
        in_specs=[
            pl.BlockSpec(memory_space=pltpu.SMEM),
            pl.BlockSpec((_D, _V), lambda i: (0, 0)),
            pl.BlockSpec((_D, _VT), lambda i: (0, i)),
            pl.BlockSpec((1, _VT), lambda i: (0, i)),
        ],
        out_specs=pl.BlockSpec((_B, _VT, _S), lambda i: (0, i, 0)),
        out_shape=jax.ShapeDtypeStruct((_B, _V, _S), jnp.float32),
        scratch_shapes=[pltpu.VMEM((_D + 1, _T), jnp.bfloat16)],
    )(ids, embT, WT, b1)

    return t.transpose(0, 2, 1)


# speedup vs baseline: 2.2384x; 1.0687x over previous
import jax
import jax.numpy as jnp
from jax import lax
from jax.experimental import pallas as pl
from jax.experimental.pallas import tpu as pltpu

_V = 100000
_D = 32
_B = 4
_S = 128
_T = _B * _S
_VT = 4096


def _body(idx_ref, et_ref, wt_ref, b_ref, o_ref, xaug_s):
    @pl.when(pl.program_id(0) == 0)
    def _gather():
        xaug_s[_D:, :] = jnp.ones((1, _T), jnp.bfloat16)
        lane_iota = lax.broadcasted_iota(jnp.int32, (1, _T), 1)

        def _tok(t, acc):
            idx = idx_ref[t // _S, t % _S]
            win = et_ref[:, pl.ds((idx // _S) * _S, _S)]
            lane = jnp.full((_D, 1), idx % _S, jnp.int32)
            col = jnp.take_along_axis(win, lane, axis=1)
            colb = jnp.broadcast_to(col, (_D, _T))
            return jnp.where(lane_iota == t, colb, acc)

        acc = lax.fori_loop(0, _T, _tok, jnp.zeros((_D, _T), jnp.float32))
        xaug_s[0:_D, :] = acc.astype(jnp.bfloat16)

    lhs = jnp.concatenate([wt_ref[...], b_ref[...]], axis=0)
    res = lax.dot_general(
        lhs.astype(jnp.bfloat16),
        xaug_s[...],
        (((0,), (0,)), ((), ())),
        preferred_element_type=jnp.float32,
    )
    for bb in range(_B):
        o_ref[bb] = res[:, bb * _S:(bb + 1) * _S]


def kernel(input_ids, emb, W, b):
    ids = input_ids.astype(jnp.int32)
    embT = emb.T
    WT = W.T
    b1 = b.reshape(1, _V)

    t = pl.pallas_call(
        _body,
        grid=(pl.cdiv(_V, _VT),),
        in_specs=[
            pl.BlockSpec(memory_space=pltpu.SMEM),
            pl.BlockSpec((_D, _V), lambda i: (0, 0)),
            pl.BlockSpec((_D, _VT), lambda i: (0, i)),
            pl.BlockSpec((1, _VT), lambda i: (0, i)),
        ],
        out_specs=pl.BlockSpec((_B, _VT, _S), lambda i: (0, i, 0)),
        out_shape=jax.ShapeDtypeStruct((_B, _V, _S), jnp.float32),
        scratch_shapes=[pltpu.VMEM((_D + 1, _T), jnp.bfloat16)],
    )(ids, embT, WT, b1)

    return t.transpose(0, 2, 1)


# speedup vs baseline: 2.2466x; 1.0037x over previous
import jax
import jax.numpy as jnp
from jax import lax
from jax.experimental import pallas as pl
from jax.experimental.pallas import tpu as pltpu

_V = 100000
_D = 32
_B = 4
_S = 128
_T = _B * _S
_VT = 8192


def _body(idx_ref, et_ref, wt_ref, b_ref, o_ref, xaug_s):
    @pl.when(pl.program_id(0) == 0)
    def _gather():
        xaug_s[_D:, :] = jnp.ones((1, _T), jnp.bfloat16)
        lane_iota = lax.broadcasted_iota(jnp.int32, (1, _T), 1)

        def _tok(t, acc):
            idx = idx_ref[t // _S, t % _S]
            win = et_ref[:, pl.ds((idx // _S) * _S, _S)]
            lane = jnp.full((_D, 1), idx % _S, jnp.int32)
            col = jnp.take_along_axis(win, lane, axis=1)
            colb = jnp.broadcast_to(col, (_D, _T))
            return jnp.where(lane_iota == t, colb, acc)

        acc = lax.fori_loop(0, _T, _tok, jnp.zeros((_D, _T), jnp.float32))
        xaug_s[0:_D, :] = acc.astype(jnp.bfloat16)

    lhs = jnp.concatenate([wt_ref[...], b_ref[...]], axis=0)
    res = lax.dot_general(
        lhs.astype(jnp.bfloat16),
        xaug_s[...],
        (((0,), (0,)), ((), ())),
        preferred_element_type=jnp.float32,
    )
    for bb in range(_B):
        o_ref[bb] = res[:, bb * _S:(bb + 1) * _S]


def kernel(input_ids, emb, W, b):
    ids = input_ids.astype(jnp.int32)
    embT = emb.T
    WT = W.T
    b1 = b.reshape(1, _V)

    t = pl.pallas_call(
        _body,
        grid=(pl.cdiv(_V, _VT),),
        in_specs=[
            pl.BlockSpec(memory_space=pltpu.SMEM),
            pl.BlockSpec((_D, _V), lambda i: (0, 0)),
            pl.BlockSpec((_D, _VT), lambda i: (0, i)),
            pl.BlockSpec((1, _VT), lambda i: (0, i)),
        ],
        out_specs=pl.BlockSpec((_B, _VT, _S), lambda i: (0, i, 0)),
        out_shape=jax.ShapeDtypeStruct((_B, _V, _S), jnp.float32),
        scratch_shapes=[pltpu.VMEM((_D + 1, _T), jnp.bfloat16)],
    )(ids, embT, WT, b1)

    return t.transpose(0, 2, 1)


# speedup vs baseline: 3.4303x; 1.5269x over previous
import jax
import jax.numpy as jnp
from jax import lax
from jax.experimental import pallas as pl
from jax.experimental.pallas import tpu as pltpu

_V = 100000
_D = 32
_B = 4
_S = 128
_T = _B * _S
_VT = 8192


def _body(idx_ref, et_ref, wt_ref, b_ref, o_ref, xaug_s):
    @pl.when(pl.program_id(0) == 0)
    def _gather():
        xaug_s[_D:, :] = jnp.ones((1, _T), jnp.bfloat16)
        lane_iota = lax.broadcasted_iota(jnp.int32, (1, _T), 1)

        def _tok(t, acc):
            idx = idx_ref[t // _S, t % _S]
            win = et_ref[:, pl.ds((idx // _S) * _S, _S)]
            lane = jnp.full((_D, 1), idx % _S, jnp.int32)
            col = jnp.take_along_axis(win, lane, axis=1)
            colb = jnp.broadcast_to(col, (_D, _T))
            return jnp.where(lane_iota == t, colb, acc)

        acc = lax.fori_loop(0, _T, _tok, jnp.zeros((_D, _T), jnp.float32), unroll=4)
        xaug_s[0:_D, :] = acc.astype(jnp.bfloat16)

    lhs = jnp.concatenate([wt_ref[...], b_ref[...]], axis=0)
    res = lax.dot_general(
        lhs.astype(jnp.bfloat16),
        xaug_s[...],
        (((0,), (0,)), ((), ())),
        preferred_element_type=jnp.float32,
    )
    for bb in range(_B):
        o_ref[bb] = res[:, bb * _S:(bb + 1) * _S]


def kernel(input_ids, emb, W, b):
    ids = input_ids.astype(jnp.int32)
    embT = emb.T
    WT = W.T
    b1 = b.reshape(1, _V)

    t = pl.pallas_call(
        _body,
        grid=(pl.cdiv(_V, _VT),),
        in_specs=[
            pl.BlockSpec(memory_space=pltpu.SMEM),
            pl.BlockSpec((_D, _V), lambda i: (0, 0)),
            pl.BlockSpec((_D, _VT), lambda i: (0, i)),
            pl.BlockSpec((1, _VT), lambda i: (0, i)),
        ],
        out_specs=pl.BlockSpec((_B, _VT, _S), lambda i: (0, i, 0)),
        out_shape=jax.ShapeDtypeStruct((_B, _V, _S), jnp.float32),
        scratch_shapes=[pltpu.VMEM((_D + 1, _T), jnp.bfloat16)],
    )(ids, embT, WT, b1)

    return t.transpose(0, 2, 1)


# speedup vs baseline: 3.7351x; 1.0889x over previous
import jax
import jax.numpy as jnp
from jax import lax
from jax.experimental import pallas as pl
from jax.experimental.pallas import tpu as pltpu

_V = 100000
_D = 32
_B = 4
_S = 128
_T = _B * _S
_VT = 8192


def _body(idx_ref, et_ref, wt_ref, b_ref, o_ref, xaug_s):
    @pl.when(pl.program_id(0) == 0)
    def _gather():
        xaug_s[_D:, :] = jnp.ones((1, _T), jnp.bfloat16)
        lane_iota = lax.broadcasted_iota(jnp.int32, (1, _T), 1)

        def _tok(t, acc):
            idx = idx_ref[t // _S, t % _S]
            win = et_ref[:, pl.ds((idx // _S) * _S, _S)]
            lane = jnp.full((_D, 1), idx % _S, jnp.int32)
            col = jnp.take_along_axis(win, lane, axis=1)
            colb = jnp.broadcast_to(col, (_D, _T))
            return jnp.where(lane_iota == t, colb, acc)

        acc = lax.fori_loop(0, _T, _tok, jnp.zeros((_D, _T), jnp.float32), unroll=8)
        xaug_s[0:_D, :] = acc.astype(jnp.bfloat16)

    lhs = jnp.concatenate([wt_ref[...], b_ref[...]], axis=0)
    res = lax.dot_general(
        lhs.astype(jnp.bfloat16),
        xaug_s[...],
        (((0,), (0,)), ((), ())),
        preferred_element_type=jnp.float32,
    )
    for bb in range(_B):
        o_ref[bb] = res[:, bb * _S:(bb + 1) * _S]


def kernel(input_ids, emb, W, b):
    ids = input_ids.astype(jnp.int32)
    embT = emb.T
    WT = W.T
    b1 = b.reshape(1, _V)

    t = pl.pallas_call(
        _body,
        grid=(pl.cdiv(_V, _VT),),
        in_specs=[
            pl.BlockSpec(memory_space=pltpu.SMEM),
            pl.BlockSpec((_D, _V), lambda i: (0, 0)),
            pl.BlockSpec((_D, _VT), lambda i: (0, i)),
            pl.BlockSpec((1, _VT), lambda i: (0, i)),
        ],
        out_specs=pl.BlockSpec((_B, _VT, _S), lambda i: (0, i, 0)),
        out_shape=jax.ShapeDtypeStruct((_B, _V, _S), jnp.float32),
        scratch_shapes=[pltpu.VMEM((_D + 1, _T), jnp.bfloat16)],
    )(ids, embT, WT, b1)

    return t.transpose(0, 2, 1)
